# baseline (speedup 1.0000x reference)
import functools

import jax
import jax.numpy as jnp
from jax import lax
from jax.experimental import pallas as pl
from jax.experimental.pallas import tpu as pltpu

_EPS = 1e-5
_INST_PER_STEP = 4


def _norm_body(x_ref, o_ref, *, d, inv_s, n_inst):
    for k in range(n_inst):
        x = x_ref[k * d:(k + 1) * d]
        p0 = jnp.sum(x, axis=0)
        q0 = jnp.sum(x * x, axis=0)
        p1 = jnp.sum(p0, axis=0, keepdims=True)
        q1 = jnp.sum(q0, axis=0, keepdims=True)
        s = jnp.sum(p1, axis=-1, keepdims=True)
        q = jnp.sum(q1, axis=-1, keepdims=True)
        mean = s * inv_s
        var = jnp.maximum(q * inv_s - mean * mean, 0.0)
        scale = lax.rsqrt(var + _EPS)
        shift = -mean * scale
        o_ref[k * d:(k + 1) * d] = x * scale + shift


def _instance_norm(x3, r, d, h, w, n_inst):
    rows = d * n_inst
    return pl.pallas_call(
        functools.partial(_norm_body, d=d, inv_s=1.0 / (d * h * w),
                          n_inst=n_inst),
        out_shape=jax.ShapeDtypeStruct(x3.shape, x3.dtype),
        grid=(r // n_inst,),
        in_specs=[pl.BlockSpec((rows, h, w), lambda i: (i, 0, 0))],
        out_specs=pl.BlockSpec((rows, h, w), lambda i: (i, 0, 0)),
        compiler_params=pltpu.CompilerParams(
            dimension_semantics=("parallel",),
        ),
    )(x3)


def kernel(x):
    n, c, d, h, w = x.shape
    r = n * c
    n_inst = _INST_PER_STEP
    while r % n_inst:
        n_inst //= 2
    x3 = x.reshape(r * d, h, w)
    out = _instance_norm(x3, r, d, h, w, n_inst)
    return out.reshape(n, c, d, h, w)


# speedup vs baseline: 1.1755x; 1.1755x over previous
import functools

import jax
import jax.numpy as jnp
from jax import lax
from jax.experimental import pallas as pl
from jax.experimental.pallas import tpu as pltpu

_EPS = 1e-5
_INST_PER_STEP = 16


def _norm_body(x_ref, o_ref, *, d, inv_s, n_inst):
    for k in range(n_inst):
        x = x_ref[k * d:(k + 1) * d]
        p0 = jnp.sum(x, axis=0)
        q0 = jnp.sum(x * x, axis=0)
        p1 = jnp.sum(p0, axis=0, keepdims=True)
        q1 = jnp.sum(q0, axis=0, keepdims=True)
        s = jnp.sum(p1, axis=-1, keepdims=True)
        q = jnp.sum(q1, axis=-1, keepdims=True)
        mean = s * inv_s
        var = jnp.maximum(q * inv_s - mean * mean, 0.0)
        scale = lax.rsqrt(var + _EPS)
        shift = -mean * scale
        o_ref[k * d:(k + 1) * d] = x * scale + shift


def _instance_norm(x3, r, d, h, w, n_inst):
    rows = d * n_inst
    return pl.pallas_call(
        functools.partial(_norm_body, d=d, inv_s=1.0 / (d * h * w),
                          n_inst=n_inst),
        out_shape=jax.ShapeDtypeStruct(x3.shape, x3.dtype),
        grid=(r // n_inst,),
        in_specs=[pl.BlockSpec((rows, h, w), lambda i: (i, 0, 0))],
        out_specs=pl.BlockSpec((rows, h, w), lambda i: (i, 0, 0)),
        compiler_params=pltpu.CompilerParams(
            dimension_semantics=("parallel",),
        ),
    )(x3)


def kernel(x):
    n, c, d, h, w = x.shape
    r = n * c
    n_inst = _INST_PER_STEP
    while r % n_inst:
        n_inst //= 2
    x3 = x.reshape(r * d, h, w)
    out = _instance_norm(x3, r, d, h, w, n_inst)
    return out.reshape(n, c, d, h, w)
